# speedup vs baseline: 1.0390x; 1.0390x over previous
import jax
import jax.numpy as jnp
from jax.experimental import pallas as pl
from jax.experimental.pallas import tpu as pltpu

N = 10000
D = 128
H = 64
HEADS = 4
L = 4

_BLK = 2000


def _final_body(nf_ref, out_ref, gmax_ref):
    p = pl.program_id(0)
    b = pl.program_id(1)

    @pl.when((p == 0) & (b == 0))
    def _():
        gmax_ref[...] = jnp.full_like(gmax_ref, -jnp.inf)

    @pl.when(p == 0)
    def _():
        m = jnp.max(nf_ref[...], axis=0, keepdims=True)
        gmax_ref[0:1, :] = jnp.maximum(gmax_ref[0:1, :], m)

    @pl.when(p == 1)
    def _():
        g = gmax_ref[0:1, :]
        out_ref[...] = jnp.concatenate(
            [nf_ref[...], jnp.broadcast_to(g, nf_ref.shape)], axis=1
        )


def _final_pool(node_feats):
    n, f = node_feats.shape
    return pl.pallas_call(
        _final_body,
        grid=(2, n // _BLK),
        in_specs=[pl.BlockSpec((_BLK, f), lambda p, b: (b, 0))],
        out_specs=pl.BlockSpec((_BLK, 2 * f), lambda p, b: (b, 0)),
        out_shape=jax.ShapeDtypeStruct((n, 2 * f), jnp.float32),
        scratch_shapes=[pltpu.VMEM((8, f), jnp.float32)],
    )(node_feats)


def kernel(x, edge_index, edge_type, rc_att, W_feat, b_feat, ET, Wrc, brc, A, bA, attn, Wn, Wm, bm):
    src = edge_index[0]
    dst = edge_index[1]
    E = src.shape[0]
    e = jnp.take(ET, edge_type, axis=0) + rc_att @ Wrc + brc
    h = jax.nn.gelu(x @ W_feat + b_feat)
    feats = [x, h]
    for l in range(L):
        hs = jnp.take(h, src, axis=0)
        hd = jnp.take(h, dst, axis=0)
        f = jnp.concatenate([hs, e, hd], axis=-1) @ A[l] + bA[l]
        f = jax.nn.leaky_relu(f, 0.2).reshape(E, HEADS, H)
        logits = jnp.sum(f * attn[l][None, :, :], axis=-1)
        ex = jnp.exp(logits)
        den = jax.ops.segment_sum(ex, dst, num_segments=N)
        alpha = ex / (jnp.take(den, dst, axis=0) + 1e-9)
        msg = (hs @ Wn[l]).reshape(E, HEADS, H) * alpha[..., None]
        agg = jax.ops.segment_sum(msg, dst, num_segments=N).reshape(N, HEADS * H)
        hn = jax.nn.gelu(agg @ Wm[l] + bm[l])
        h = hn + h
        feats.append(h)
    node_feats = jnp.concatenate(feats, axis=-1)
    return _final_pool(node_feats)


# speedup vs baseline: 29.6897x; 28.5747x over previous
import functools

import jax
import jax.numpy as jnp
from jax import lax
from jax.experimental import pallas as pl
from jax.experimental.pallas import tpu as pltpu
from jax.experimental.pallas import tpu_sc as plsc

N = 10000
E = 320000
D = 128
H = 64
HEADS = 4
LAYERS = 4
NET = 41
NETP = 48

T = 1280
G = E // T
W = 128

NW = 32
CHUNK = 128
CPW = 80
E_PAD = NW * CPW * CHUNK
NBUF = 4

f32 = jnp.float32
bf16 = jnp.bfloat16
i32 = jnp.int32


def _sc_gather_body(h16_hbm, idx_hbm, out_hbm, idx_v, rows_v, gsem):
    cid = lax.axis_index("c")
    sid = lax.axis_index("s")
    wid = sid * 2 + cid
    pltpu.sync_copy(idx_hbm.at[pl.ds(wid * CPW, CPW)], idx_v)

    def start(j, b):
        pltpu.async_copy(h16_hbm.at[idx_v.at[j]], rows_v.at[b], gsem.at[b])

    def wait(j, b):
        pltpu.make_async_copy(
            h16_hbm.at[idx_v.at[j]], rows_v.at[b], gsem.at[b]
        ).wait()

    for b in range(NBUF):
        start(b, b)

    @pl.loop(0, CPW, step=NBUF)
    def _(jj):
        for b in range(NBUF):
            j = jj + b
            wait(j, b)
            c = wid * CPW + j
            pltpu.sync_copy(rows_v.at[b], out_hbm.at[pl.ds(c * CHUNK, CHUNK)])

            @pl.when(j + NBUF < CPW)
            def _():
                start(j + NBUF, b)


def _sc_gather(h_pad, idx2d):
    mesh = plsc.VectorSubcoreMesh(core_axis_name="c", subcore_axis_name="s")
    kern = pl.kernel(
        _sc_gather_body,
        out_type=jax.ShapeDtypeStruct((E_PAD, D), f32),
        mesh=mesh,
        scratch_types=[
            pltpu.VMEM((CPW, CHUNK), i32),
            pltpu.VMEM((NBUF, CHUNK, D), f32),
            pltpu.SemaphoreType.DMA((NBUF,)),
        ],
    )
    return kern(h_pad, idx2d)


def _prolog_body(x_ref, wf_ref, bf_ref, h_ref, h16_ref):
    h = jnp.dot(x_ref[...], wf_ref[...], preferred_element_type=f32)
    h = jax.nn.gelu(h + bf_ref[...])
    h_ref[:, 0:H] = h
    h_ref[:, H:D] = jnp.zeros((h.shape[0], D - H), f32)
    h16_ref[...] = h.astype(bf16)


def _prolog(x, W_feat, b_feat):
    blk = 2000
    return pl.pallas_call(
        _prolog_body,
        grid=(N // blk,),
        in_specs=[
            pl.BlockSpec((blk, D), lambda b: (b, 0)),
            pl.BlockSpec((D, H), lambda b: (0, 0)),
            pl.BlockSpec((1, H), lambda b: (0, 0)),
        ],
        out_specs=[
            pl.BlockSpec((blk, D), lambda b: (b, 0)),
            pl.BlockSpec((blk, H), lambda b: (b, 0)),
        ],
        out_shape=[
            jax.ShapeDtypeStruct((N, D), f32),
            jax.ShapeDtypeStruct((N, H), bf16),
        ],
    )(x, W_feat, b_feat.reshape(1, H))


def _edge_body(bases_ref, hs_ref, et_ref, dst_ref, rc0_ref, rc1_ref, h16_ref,
               ET_ref, Wrc_ref, brc_ref, A1_ref, A2_ref, A3_ref, bA_ref,
               attn_ref, ones_ref, z0_ref, z1_ref, z2_ref, z3_ref, den_ref):
    g = pl.program_id(0)

    @pl.when(g == 0)
    def _():
        z0_ref[...] = jnp.zeros_like(z0_ref)
        z1_ref[...] = jnp.zeros_like(z1_ref)
        z2_ref[...] = jnp.zeros_like(z2_ref)
        z3_ref[...] = jnp.zeros_like(z3_ref)
        den_ref[...] = jnp.zeros_like(den_ref)

    base = pl.multiple_of(bases_ref[g], 16)
    dstr = dst_ref[0]
    etr = et_ref[0]
    hsf = hs_ref[:, 0:H]
    hs16 = hsf.astype(bf16)

    ids48 = lax.broadcasted_iota(i32, (NETP, T), 0)
    OH = (ids48 == etr).astype(bf16)
    rc0 = rc0_ref[0].astype(bf16)
    rc1 = rc1_ref[0].astype(bf16)
    RC = jnp.concatenate([rc0, rc1, jnp.zeros((6, T), bf16)], axis=0)
    cdn = (((0,), (0,)), ((), ()))
    e_f = lax.dot_general(OH, ET_ref[...], cdn, preferred_element_type=f32)
    e_f = e_f + lax.dot_general(RC, Wrc_ref[...], cdn, preferred_element_type=f32)
    e16 = (e_f + brc_ref[...]).astype(bf16)

    idsW = lax.broadcasted_iota(i32, (W, T), 0) + base
    PT = (idsW == dstr).astype(bf16)

    h_win = h16_ref[pl.ds(base, W), :]
    hd16 = lax.dot_general(PT, h_win, cdn, preferred_element_type=f32).astype(bf16)

    f = jnp.dot(hs16, A1_ref[...], preferred_element_type=f32)
    f = f + jnp.dot(e16, A2_ref[...], preferred_element_type=f32)
    f = f + jnp.dot(hd16, A3_ref[...], preferred_element_type=f32)
    f = f + bA_ref[...]
    f = jnp.where(f >= 0, f, 0.2 * f)

    fa16 = (f * attn_ref[...]).astype(bf16)
    lg = jnp.dot(fa16, ones_ref[...], preferred_element_type=f32)
    ex = jnp.exp(lg)

    den_ref[pl.ds(base, W), :] += jnp.dot(
        PT, ex.astype(bf16), preferred_element_type=f32)

    for k, zr in enumerate((z0_ref, z1_ref, z2_ref, z3_ref)):
        pk = (hsf * ex[:, k:k + 1]).astype(bf16)
        zr[pl.ds(base, W), :] += jnp.dot(PT, pk, preferred_element_type=f32)


def _edge(bases, hs_pad, et3, dst3, rc03, rc13, h16, ET48, Wrc8, brc_r,
          A1l, A2l, A3l, bAl, attnl, ones_blk):
    grid_spec = pltpu.PrefetchScalarGridSpec(
        num_scalar_prefetch=1,
        grid=(G,),
        in_specs=[
            pl.BlockSpec((T, D), lambda g, b: (g, 0)),
            pl.BlockSpec((1, 1, T), lambda g, b: (g, 0, 0)),
            pl.BlockSpec((1, 1, T), lambda g, b: (g, 0, 0)),
            pl.BlockSpec((1, 1, T), lambda g, b: (g, 0, 0)),
            pl.BlockSpec((1, 1, T), lambda g, b: (g, 0, 0)),
            pl.BlockSpec((N, H), lambda g, b: (0, 0)),
            pl.BlockSpec((NETP, H), lambda g, b: (0, 0)),
            pl.BlockSpec((8, H), lambda g, b: (0, 0)),
            pl.BlockSpec((1, H), lambda g, b: (0, 0)),
            pl.BlockSpec((H, 4 * H), lambda g, b: (0, 0)),
            pl.BlockSpec((H, 4 * H), lambda g, b: (0, 0)),
            pl.BlockSpec((H, 4 * H), lambda g, b: (0, 0)),
            pl.BlockSpec((1, 4 * H), lambda g, b: (0, 0)),
            pl.BlockSpec((1, 4 * H), lambda g, b: (0, 0)),
            pl.BlockSpec((4 * H, 8), lambda g, b: (0, 0)),
        ],
        out_specs=[
            pl.BlockSpec((N, H), lambda g, b: (0, 0)),
            pl.BlockSpec((N, H), lambda g, b: (0, 0)),
            pl.BlockSpec((N, H), lambda g, b: (0, 0)),
            pl.BlockSpec((N, H), lambda g, b: (0, 0)),
            pl.BlockSpec((N, 8), lambda g, b: (0, 0)),
        ],
    )
    return pl.pallas_call(
        _edge_body,
        grid_spec=grid_spec,
        out_shape=[jax.ShapeDtypeStruct((N, H), f32)] * 4
        + [jax.ShapeDtypeStruct((N, 8), f32)],
    )(bases, hs_pad, et3, dst3, rc03, rc13, h16, ET48, Wrc8, brc_r,
      A1l, A2l, A3l, bAl, attnl, ones_blk)


def _node_body(z0_ref, z1_ref, z2_ref, z3_ref, den_ref, h_ref, WW_ref,
               bm_ref, h_out_ref, h16_out_ref):
    zs = (z0_ref, z1_ref, z2_ref, z3_ref)
    u = None
    for k in range(HEADS):
        r = 1.0 / (den_ref[:, k:k + 1] + 1e-9)
        q = zs[k][...] * r
        t = jnp.dot(q, WW_ref[k], preferred_element_type=f32)
        u = t if u is None else u + t
    hn = jax.nn.gelu(u + bm_ref[...])
    h = hn + h_ref[:, 0:H]
    h_out_ref[:, 0:H] = h
    h_out_ref[:, H:D] = jnp.zeros((h.shape[0], D - H), f32)
    h16_out_ref[...] = h.astype(bf16)


def _node(z0, z1, z2, z3, den, h, WWl, bm_r):
    blk = 2000
    return pl.pallas_call(
        _node_body,
        grid=(N // blk,),
        in_specs=[
            pl.BlockSpec((blk, H), lambda b: (b, 0)),
            pl.BlockSpec((blk, H), lambda b: (b, 0)),
            pl.BlockSpec((blk, H), lambda b: (b, 0)),
            pl.BlockSpec((blk, H), lambda b: (b, 0)),
            pl.BlockSpec((blk, 8), lambda b: (b, 0)),
            pl.BlockSpec((blk, D), lambda b: (b, 0)),
            pl.BlockSpec((HEADS, H, H), lambda b: (0, 0, 0)),
            pl.BlockSpec((1, H), lambda b: (0, 0)),
        ],
        out_specs=[
            pl.BlockSpec((blk, D), lambda b: (b, 0)),
            pl.BlockSpec((blk, H), lambda b: (b, 0)),
        ],
        out_shape=[
            jax.ShapeDtypeStruct((N, D), f32),
            jax.ShapeDtypeStruct((N, H), bf16),
        ],
    )(z0, z1, z2, z3, den, h, WWl, bm_r)


_NF = D + (LAYERS + 1) * H


def _final_body(x_ref, h0_ref, h1_ref, h2_ref, h3_ref, h4_ref, out_ref,
                gmax_ref):
    p = pl.program_id(0)
    b = pl.program_id(1)
    hs = (h0_ref, h1_ref, h2_ref, h3_ref, h4_ref)

    @pl.when((p == 0) & (b == 0))
    def _():
        gmax_ref[...] = jnp.full_like(gmax_ref, -jnp.inf)

    @pl.when(p == 0)
    def _():
        m = jnp.max(x_ref[...], axis=0, keepdims=True)
        gmax_ref[0:1, 0:D] = jnp.maximum(gmax_ref[0:1, 0:D], m)
        for i, hr in enumerate(hs):
            lo = D + i * H
            m = jnp.max(hr[:, 0:H], axis=0, keepdims=True)
            gmax_ref[0:1, lo:lo + H] = jnp.maximum(gmax_ref[0:1, lo:lo + H], m)

    @pl.when(p == 1)
    def _():
        out_ref[:, 0:D] = x_ref[...]
        for i, hr in enumerate(hs):
            lo = D + i * H
            out_ref[:, lo:lo + H] = hr[:, 0:H]
        g = gmax_ref[0:1, 0:_NF]
        out_ref[:, _NF:2 * _NF] = jnp.broadcast_to(g, (x_ref.shape[0], _NF))


def _final(x, h0, h1, h2, h3, h4):
    blk = 2000
    return pl.pallas_call(
        _final_body,
        grid=(2, N // blk),
        in_specs=[pl.BlockSpec((blk, D), lambda p, b: (b, 0))] * 6,
        out_specs=pl.BlockSpec((blk, 2 * _NF), lambda p, b: (b, 0)),
        out_shape=jax.ShapeDtypeStruct((N, 2 * _NF), f32),
        scratch_shapes=[pltpu.VMEM((8, _NF), f32)],
    )(x, h0, h1, h2, h3, h4)


def kernel(x, edge_index, edge_type, rc_att, W_feat, b_feat, ET, Wrc, brc, A,
           bA, attn, Wn, Wm, bm):
    src = edge_index[0]
    dst = edge_index[1]

    dst_s, perm = lax.sort_key_val(dst, jnp.arange(E, dtype=i32))
    src_s = jnp.take(src, perm)
    et_s = jnp.take(edge_type, perm)
    rc_s = jnp.take(rc_att, perm, axis=0)

    bases = jnp.minimum((dst_s[::T] // 16) * 16, N - W).astype(i32)

    et3 = et_s.reshape(G, 1, T)
    dst3 = dst_s.reshape(G, 1, T)
    rc03 = rc_s[:, 0].reshape(G, 1, T)
    rc13 = rc_s[:, 1].reshape(G, 1, T)
    src_pad = jnp.concatenate([src_s, jnp.zeros(E_PAD - E, i32)])
    idx2d = src_pad.reshape(E_PAD // CHUNK, CHUNK)

    ET48 = jnp.concatenate([ET, jnp.zeros((NETP - NET, H), f32)]).astype(bf16)
    Wrc8 = jnp.concatenate([Wrc, jnp.zeros((6, H), f32)]).astype(bf16)
    brc_r = brc.reshape(1, H)
    A1 = A[:, 0:H, :].astype(bf16)
    A2 = A[:, H:2 * H, :].astype(bf16)
    A3 = A[:, 2 * H:3 * H, :].astype(bf16)
    bA_r = bA.reshape(LAYERS, 1, HEADS * H)
    attn_r = attn.reshape(LAYERS, 1, HEADS * H)
    col = jnp.arange(HEADS * H) // H
    ones_blk = (col[:, None] == jnp.arange(8)[None, :]).astype(bf16)
    Wn4 = Wn.reshape(LAYERS, H, HEADS, H).transpose(0, 2, 1, 3)
    Wm4 = Wm.reshape(LAYERS, HEADS, H, H)
    WW = jnp.einsum("lkab,lkbc->lkac", Wn4, Wm4)
    bm_r = bm.reshape(LAYERS, 1, H)

    h, h16 = _prolog(x, W_feat, b_feat)
    feats = [h]

    for l in range(LAYERS):
        hs_pad = _sc_gather(h, idx2d)
        z0, z1, z2, z3, den = _edge(
            bases, hs_pad, et3, dst3, rc03, rc13, h16, ET48, Wrc8, brc_r,
            A1[l], A2[l], A3[l], bA_r[l], attn_r[l], ones_blk)
        h, h16 = _node(z0, z1, z2, z3, den, h, WW[l], bm_r[l])
        feats.append(h)

    return _final(x, *feats)


# speedup vs baseline: 29.6952x; 1.0002x over previous
import functools

import jax
import jax.numpy as jnp
from jax import lax
from jax.experimental import pallas as pl
from jax.experimental.pallas import tpu as pltpu
from jax.experimental.pallas import tpu_sc as plsc

N = 10000
E = 320000
D = 128
H = 64
HEADS = 4
LAYERS = 4
NET = 41
NETP = 48

T = 1280
G = E // T
W = 128

NW = 32
CHUNK = 128
CPW = 80
E_PAD = NW * CPW * CHUNK
NBUF = 6

f32 = jnp.float32
bf16 = jnp.bfloat16
i32 = jnp.int32


K_INFLIGHT = 3
SC_TOT = 84


def _sc_gather_body(h16_hbm, idx_hbm, out_hbm, idx_v, rows_v, gsem, wsem):
    cid = lax.axis_index("c")
    sid = lax.axis_index("s")
    wid = sid * 2 + cid
    pltpu.sync_copy(idx_hbm.at[pl.ds(wid * CPW, CPW)], idx_v)

    def start_g(j, b):
        pltpu.async_copy(h16_hbm.at[idx_v.at[j]], rows_v.at[b], gsem.at[b])

    def wait_g(j, b):
        pltpu.make_async_copy(
            h16_hbm.at[idx_v.at[j]], rows_v.at[b], gsem.at[b]
        ).wait()

    def start_w(j, b):
        c = wid * CPW + j
        pltpu.async_copy(rows_v.at[b], out_hbm.at[pl.ds(c * CHUNK, CHUNK)],
                         wsem.at[b])

    def wait_w(j, b):
        c = wid * CPW + j
        pltpu.make_async_copy(
            rows_v.at[b], out_hbm.at[pl.ds(c * CHUNK, CHUNK)], wsem.at[b]
        ).wait()

    @pl.loop(0, SC_TOT, step=NBUF)
    def _(ii):
        for db in range(NBUF):
            i = ii + db
            b = db
            @pl.when((i >= NBUF) & (i < CPW))
            def _():
                wait_w(i - NBUF, b)

            @pl.when(i < CPW)
            def _():
                start_g(i, b)

            j = i - K_INFLIGHT
            bj = (db - K_INFLIGHT) % NBUF

            @pl.when((j >= 0) & (j < CPW))
            def _():
                wait_g(j, bj)
                start_w(j, bj)

    for d in range(NBUF):
        j = CPW - NBUF + d
        wait_w(j, j % NBUF)


def _sc_gather(h_pad, idx2d):
    mesh = plsc.VectorSubcoreMesh(core_axis_name="c", subcore_axis_name="s")
    kern = pl.kernel(
        _sc_gather_body,
        out_type=jax.ShapeDtypeStruct((E_PAD, D), f32),
        mesh=mesh,
        scratch_types=[
            pltpu.VMEM((CPW, CHUNK), i32),
            pltpu.VMEM((NBUF, CHUNK, D), f32),
            pltpu.SemaphoreType.DMA((NBUF,)),
            pltpu.SemaphoreType.DMA((NBUF,)),
        ],
    )
    return kern(h_pad, idx2d)


def _prolog_body(x_ref, wf_ref, bf_ref, h_ref, h16_ref):
    h = jnp.dot(x_ref[...], wf_ref[...], preferred_element_type=f32)
    h = jax.nn.gelu(h + bf_ref[...])
    h_ref[:, 0:H] = h
    h_ref[:, H:D] = jnp.zeros((h.shape[0], D - H), f32)
    h16_ref[...] = h.astype(bf16)


def _prolog(x, W_feat, b_feat):
    blk = 2000
    return pl.pallas_call(
        _prolog_body,
        grid=(N // blk,),
        in_specs=[
            pl.BlockSpec((blk, D), lambda b: (b, 0)),
            pl.BlockSpec((D, H), lambda b: (0, 0)),
            pl.BlockSpec((1, H), lambda b: (0, 0)),
        ],
        out_specs=[
            pl.BlockSpec((blk, D), lambda b: (b, 0)),
            pl.BlockSpec((blk, H), lambda b: (b, 0)),
        ],
        out_shape=[
            jax.ShapeDtypeStruct((N, D), f32),
            jax.ShapeDtypeStruct((N, H), bf16),
        ],
    )(x, W_feat, b_feat.reshape(1, H))


def _edge_body(bases_ref, hs_ref, et_ref, dst_ref, rc0_ref, rc1_ref, h16_ref,
               ET_ref, Wrc_ref, brc_ref, A1_ref, A2_ref, A3_ref, bA_ref,
               attn_ref, ones_ref, z0_ref, z1_ref, z2_ref, z3_ref, den_ref):
    g = pl.program_id(0)

    @pl.when(g == 0)
    def _():
        z0_ref[...] = jnp.zeros_like(z0_ref)
        z1_ref[...] = jnp.zeros_like(z1_ref)
        z2_ref[...] = jnp.zeros_like(z2_ref)
        z3_ref[...] = jnp.zeros_like(z3_ref)
        den_ref[...] = jnp.zeros_like(den_ref)

    base = pl.multiple_of(bases_ref[g], 16)
    dstr = dst_ref[0]
    etr = et_ref[0]
    hsf = hs_ref[:, 0:H]
    hs16 = hsf.astype(bf16)

    ids48 = lax.broadcasted_iota(i32, (NETP, T), 0)
    OH = (ids48 == etr).astype(bf16)
    rc0 = rc0_ref[0].astype(bf16)
    rc1 = rc1_ref[0].astype(bf16)
    RC = jnp.concatenate([rc0, rc1, jnp.zeros((6, T), bf16)], axis=0)
    cdn = (((0,), (0,)), ((), ()))
    e_f = lax.dot_general(OH, ET_ref[...], cdn, preferred_element_type=f32)
    e_f = e_f + lax.dot_general(RC, Wrc_ref[...], cdn, preferred_element_type=f32)
    e16 = (e_f + brc_ref[...]).astype(bf16)

    idsW = lax.broadcasted_iota(i32, (W, T), 0) + base
    PT = (idsW == dstr).astype(bf16)

    h_win = h16_ref[pl.ds(base, W), :]
    hd16 = lax.dot_general(PT, h_win, cdn, preferred_element_type=f32).astype(bf16)

    f = jnp.dot(hs16, A1_ref[...], preferred_element_type=f32)
    f = f + jnp.dot(e16, A2_ref[...], preferred_element_type=f32)
    f = f + jnp.dot(hd16, A3_ref[...], preferred_element_type=f32)
    f = f + bA_ref[...]
    f = jnp.where(f >= 0, f, 0.2 * f)

    fa16 = (f * attn_ref[...]).astype(bf16)
    lg = jnp.dot(fa16, ones_ref[...], preferred_element_type=f32)
    ex = jnp.exp(lg)

    den_ref[pl.ds(base, W), :] += jnp.dot(
        PT, ex.astype(bf16), preferred_element_type=f32)

    for k, zr in enumerate((z0_ref, z1_ref, z2_ref, z3_ref)):
        pk = (hsf * ex[:, k:k + 1]).astype(bf16)
        zr[pl.ds(base, W), :] += jnp.dot(PT, pk, preferred_element_type=f32)


def _edge(bases, hs_pad, et3, dst3, rc03, rc13, h16, ET48, Wrc8, brc_r,
          A1l, A2l, A3l, bAl, attnl, ones_blk):
    grid_spec = pltpu.PrefetchScalarGridSpec(
        num_scalar_prefetch=1,
        grid=(G,),
        in_specs=[
            pl.BlockSpec((T, D), lambda g, b: (g, 0)),
            pl.BlockSpec((1, 1, T), lambda g, b: (g, 0, 0)),
            pl.BlockSpec((1, 1, T), lambda g, b: (g, 0, 0)),
            pl.BlockSpec((1, 1, T), lambda g, b: (g, 0, 0)),
            pl.BlockSpec((1, 1, T), lambda g, b: (g, 0, 0)),
            pl.BlockSpec((N, H), lambda g, b: (0, 0)),
            pl.BlockSpec((NETP, H), lambda g, b: (0, 0)),
            pl.BlockSpec((8, H), lambda g, b: (0, 0)),
            pl.BlockSpec((1, H), lambda g, b: (0, 0)),
            pl.BlockSpec((H, 4 * H), lambda g, b: (0, 0)),
            pl.BlockSpec((H, 4 * H), lambda g, b: (0, 0)),
            pl.BlockSpec((H, 4 * H), lambda g, b: (0, 0)),
            pl.BlockSpec((1, 4 * H), lambda g, b: (0, 0)),
            pl.BlockSpec((1, 4 * H), lambda g, b: (0, 0)),
            pl.BlockSpec((4 * H, 8), lambda g, b: (0, 0)),
        ],
        out_specs=[
            pl.BlockSpec((N, H), lambda g, b: (0, 0)),
            pl.BlockSpec((N, H), lambda g, b: (0, 0)),
            pl.BlockSpec((N, H), lambda g, b: (0, 0)),
            pl.BlockSpec((N, H), lambda g, b: (0, 0)),
            pl.BlockSpec((N, 8), lambda g, b: (0, 0)),
        ],
    )
    return pl.pallas_call(
        _edge_body,
        grid_spec=grid_spec,
        out_shape=[jax.ShapeDtypeStruct((N, H), f32)] * 4
        + [jax.ShapeDtypeStruct((N, 8), f32)],
    )(bases, hs_pad, et3, dst3, rc03, rc13, h16, ET48, Wrc8, brc_r,
      A1l, A2l, A3l, bAl, attnl, ones_blk)


def _node_body(z0_ref, z1_ref, z2_ref, z3_ref, den_ref, h_ref, WW_ref,
               bm_ref, h_out_ref, h16_out_ref):
    zs = (z0_ref, z1_ref, z2_ref, z3_ref)
    u = None
    for k in range(HEADS):
        r = 1.0 / (den_ref[:, k:k + 1] + 1e-9)
        q = zs[k][...] * r
        t = jnp.dot(q, WW_ref[k], preferred_element_type=f32)
        u = t if u is None else u + t
    hn = jax.nn.gelu(u + bm_ref[...])
    h = hn + h_ref[:, 0:H]
    h_out_ref[:, 0:H] = h
    h_out_ref[:, H:D] = jnp.zeros((h.shape[0], D - H), f32)
    h16_out_ref[...] = h.astype(bf16)


def _node(z0, z1, z2, z3, den, h, WWl, bm_r):
    blk = 2000
    return pl.pallas_call(
        _node_body,
        grid=(N // blk,),
        in_specs=[
            pl.BlockSpec((blk, H), lambda b: (b, 0)),
            pl.BlockSpec((blk, H), lambda b: (b, 0)),
            pl.BlockSpec((blk, H), lambda b: (b, 0)),
            pl.BlockSpec((blk, H), lambda b: (b, 0)),
            pl.BlockSpec((blk, 8), lambda b: (b, 0)),
            pl.BlockSpec((blk, D), lambda b: (b, 0)),
            pl.BlockSpec((HEADS, H, H), lambda b: (0, 0, 0)),
            pl.BlockSpec((1, H), lambda b: (0, 0)),
        ],
        out_specs=[
            pl.BlockSpec((blk, D), lambda b: (b, 0)),
            pl.BlockSpec((blk, H), lambda b: (b, 0)),
        ],
        out_shape=[
            jax.ShapeDtypeStruct((N, D), f32),
            jax.ShapeDtypeStruct((N, H), bf16),
        ],
    )(z0, z1, z2, z3, den, h, WWl, bm_r)


_NF = D + (LAYERS + 1) * H


def _final_body(x_ref, h0_ref, h1_ref, h2_ref, h3_ref, h4_ref, out_ref,
                gmax_ref):
    p = pl.program_id(0)
    b = pl.program_id(1)
    hs = (h0_ref, h1_ref, h2_ref, h3_ref, h4_ref)

    @pl.when((p == 0) & (b == 0))
    def _():
        gmax_ref[...] = jnp.full_like(gmax_ref, -jnp.inf)

    @pl.when(p == 0)
    def _():
        m = jnp.max(x_ref[...], axis=0, keepdims=True)
        gmax_ref[0:1, 0:D] = jnp.maximum(gmax_ref[0:1, 0:D], m)
        for i, hr in enumerate(hs):
            lo = D + i * H
            m = jnp.max(hr[:, 0:H], axis=0, keepdims=True)
            gmax_ref[0:1, lo:lo + H] = jnp.maximum(gmax_ref[0:1, lo:lo + H], m)

    @pl.when(p == 1)
    def _():
        out_ref[:, 0:D] = x_ref[...]
        for i, hr in enumerate(hs):
            lo = D + i * H
            out_ref[:, lo:lo + H] = hr[:, 0:H]
        g = gmax_ref[0:1, 0:_NF]
        out_ref[:, _NF:2 * _NF] = jnp.broadcast_to(g, (x_ref.shape[0], _NF))


def _final(x, h0, h1, h2, h3, h4):
    blk = 2000
    return pl.pallas_call(
        _final_body,
        grid=(2, N // blk),
        in_specs=[pl.BlockSpec((blk, D), lambda p, b: (b, 0))] * 6,
        out_specs=pl.BlockSpec((blk, 2 * _NF), lambda p, b: (b, 0)),
        out_shape=jax.ShapeDtypeStruct((N, 2 * _NF), f32),
        scratch_shapes=[pltpu.VMEM((8, _NF), f32)],
    )(x, h0, h1, h2, h3, h4)


def kernel(x, edge_index, edge_type, rc_att, W_feat, b_feat, ET, Wrc, brc, A,
           bA, attn, Wn, Wm, bm):
    src = edge_index[0]
    dst = edge_index[1]

    dst_s, perm = lax.sort_key_val(dst, jnp.arange(E, dtype=i32))
    src_s = jnp.take(src, perm)
    et_s = jnp.take(edge_type, perm)
    rc_s = jnp.take(rc_att, perm, axis=0)

    bases = jnp.minimum((dst_s[::T] // 16) * 16, N - W).astype(i32)

    et3 = et_s.reshape(G, 1, T)
    dst3 = dst_s.reshape(G, 1, T)
    rc03 = rc_s[:, 0].reshape(G, 1, T)
    rc13 = rc_s[:, 1].reshape(G, 1, T)
    src_pad = jnp.concatenate([src_s, jnp.zeros(E_PAD - E, i32)])
    idx2d = src_pad.reshape(E_PAD // CHUNK, CHUNK)

    ET48 = jnp.concatenate([ET, jnp.zeros((NETP - NET, H), f32)]).astype(bf16)
    Wrc8 = jnp.concatenate([Wrc, jnp.zeros((6, H), f32)]).astype(bf16)
    brc_r = brc.reshape(1, H)
    A1 = A[:, 0:H, :].astype(bf16)
    A2 = A[:, H:2 * H, :].astype(bf16)
    A3 = A[:, 2 * H:3 * H, :].astype(bf16)
    bA_r = bA.reshape(LAYERS, 1, HEADS * H)
    attn_r = attn.reshape(LAYERS, 1, HEADS * H)
    col = jnp.arange(HEADS * H) // H
    ones_blk = (col[:, None] == jnp.arange(8)[None, :]).astype(bf16)
    Wn4 = Wn.reshape(LAYERS, H, HEADS, H).transpose(0, 2, 1, 3)
    Wm4 = Wm.reshape(LAYERS, HEADS, H, H)
    WW = jnp.einsum("lkab,lkbc->lkac", Wn4, Wm4)
    bm_r = bm.reshape(LAYERS, 1, H)

    h, h16 = _prolog(x, W_feat, b_feat)
    feats = [h]

    for l in range(LAYERS):
        hs_pad = _sc_gather(h, idx2d)
        z0, z1, z2, z3, den = _edge(
            bases, hs_pad, et3, dst3, rc03, rc13, h16, ET48, Wrc8, brc_r,
            A1[l], A2[l], A3[l], bA_r[l], attn_r[l], ones_blk)
        h, h16 = _node(z0, z1, z2, z3, den, h, WW[l], bm_r[l])
        feats.append(h)

    return _final(x, *feats)


# speedup vs baseline: 37.6468x; 1.2678x over previous
import functools

import jax
import jax.numpy as jnp
from jax import lax
from jax.experimental import pallas as pl
from jax.experimental.pallas import tpu as pltpu
from jax.experimental.pallas import tpu_sc as plsc

N = 10000
E = 320000
D = 128
H = 64
HEADS = 4
LAYERS = 4
NET = 41
NETP = 48

T = 1280
G = E // T
W = 128

NW = 32
CHUNK = 128
CPW = 80
E_PAD = NW * CPW * CHUNK
NBUF = 6

f32 = jnp.float32
bf16 = jnp.bfloat16
i32 = jnp.int32


K_INFLIGHT = 3
SC_TOT = 84


def _sc_gather_body(h16_hbm, idx_hbm, out_hbm, idx_v, rows_v, gsem, wsem):
    cid = lax.axis_index("c")
    sid = lax.axis_index("s")
    wid = sid * 2 + cid
    pltpu.sync_copy(idx_hbm.at[pl.ds(wid * CPW, CPW)], idx_v)

    def start_g(j, b):
        pltpu.async_copy(h16_hbm.at[idx_v.at[j]], rows_v.at[b], gsem.at[b])

    def wait_g(j, b):
        pltpu.make_async_copy(
            h16_hbm.at[idx_v.at[j]], rows_v.at[b], gsem.at[b]
        ).wait()

    def start_w(j, b):
        c = wid * CPW + j
        pltpu.async_copy(rows_v.at[b], out_hbm.at[pl.ds(c * CHUNK, CHUNK)],
                         wsem.at[b])

    def wait_w(j, b):
        c = wid * CPW + j
        pltpu.make_async_copy(
            rows_v.at[b], out_hbm.at[pl.ds(c * CHUNK, CHUNK)], wsem.at[b]
        ).wait()

    @pl.loop(0, SC_TOT, step=NBUF)
    def _(ii):
        for db in range(NBUF):
            i = ii + db
            b = db
            @pl.when((i >= NBUF) & (i < CPW))
            def _():
                wait_w(i - NBUF, b)

            @pl.when(i < CPW)
            def _():
                start_g(i, b)

            j = i - K_INFLIGHT
            bj = (db - K_INFLIGHT) % NBUF

            @pl.when((j >= 0) & (j < CPW))
            def _():
                wait_g(j, bj)
                start_w(j, bj)

    for d in range(NBUF):
        j = CPW - NBUF + d
        wait_w(j, j % NBUF)


def _sc_gather(h_pad, idx2d):
    mesh = plsc.VectorSubcoreMesh(core_axis_name="c", subcore_axis_name="s")
    kern = pl.kernel(
        _sc_gather_body,
        out_type=jax.ShapeDtypeStruct((E_PAD, D), f32),
        mesh=mesh,
        scratch_types=[
            pltpu.VMEM((CPW, CHUNK), i32),
            pltpu.VMEM((NBUF, CHUNK, D), f32),
            pltpu.SemaphoreType.DMA((NBUF,)),
            pltpu.SemaphoreType.DMA((NBUF,)),
        ],
    )
    return kern(h_pad, idx2d)


def _prolog_body(x_ref, wf_ref, bf_ref, h_ref, h16_ref):
    h = jnp.dot(x_ref[...], wf_ref[...], preferred_element_type=f32)
    h = jax.nn.gelu(h + bf_ref[...])
    h_ref[:, 0:H] = h
    h_ref[:, H:D] = jnp.zeros((h.shape[0], D - H), f32)
    h16_ref[...] = h.astype(bf16)


def _prolog(x, W_feat, b_feat):
    blk = 2000
    return pl.pallas_call(
        _prolog_body,
        grid=(N // blk,),
        in_specs=[
            pl.BlockSpec((blk, D), lambda b: (b, 0)),
            pl.BlockSpec((D, H), lambda b: (0, 0)),
            pl.BlockSpec((1, H), lambda b: (0, 0)),
        ],
        out_specs=[
            pl.BlockSpec((blk, D), lambda b: (b, 0)),
            pl.BlockSpec((blk, H), lambda b: (b, 0)),
        ],
        out_shape=[
            jax.ShapeDtypeStruct((N, D), f32),
            jax.ShapeDtypeStruct((N, H), bf16),
        ],
    )(x, W_feat, b_feat.reshape(1, H))


def _embed_body(et_ref, rc0_ref, rc1_ref, ET_ref, Wrc_ref, brc_ref, e_ref):
    etr = et_ref[0]
    ids48 = lax.broadcasted_iota(i32, (NETP, T), 0)
    OH = (ids48 == etr).astype(bf16)
    rc0 = rc0_ref[0].astype(bf16)
    rc1 = rc1_ref[0].astype(bf16)
    RC = jnp.concatenate([rc0, rc1, jnp.zeros((6, T), bf16)], axis=0)
    cdn = (((0,), (0,)), ((), ()))
    e_f = lax.dot_general(OH, ET_ref[...], cdn, preferred_element_type=f32)
    e_f = e_f + lax.dot_general(RC, Wrc_ref[...], cdn, preferred_element_type=f32)
    e_ref[...] = (e_f + brc_ref[...]).astype(bf16)


def _embed(et3, rc03, rc13, ET48, Wrc8, brc_r):
    return pl.pallas_call(
        _embed_body,
        grid=(G,),
        in_specs=[
            pl.BlockSpec((1, 1, T), lambda g: (g, 0, 0)),
            pl.BlockSpec((1, 1, T), lambda g: (g, 0, 0)),
            pl.BlockSpec((1, 1, T), lambda g: (g, 0, 0)),
            pl.BlockSpec((NETP, H), lambda g: (0, 0)),
            pl.BlockSpec((8, H), lambda g: (0, 0)),
            pl.BlockSpec((1, H), lambda g: (0, 0)),
        ],
        out_specs=pl.BlockSpec((T, H), lambda g: (g, 0)),
        out_shape=jax.ShapeDtypeStruct((E, H), bf16),
    )(et3, rc03, rc13, ET48, Wrc8, brc_r)


def _edge_body(bases_ref, hs_ref, e_ref, dst_ref, h16_ref,
               A192_ref, bA_ref, attn_ref, ones_ref, exp8_ref,
               z_ref, den_ref):
    g = pl.program_id(0)

    @pl.when(g == 0)
    def _():
        z_ref[...] = jnp.zeros_like(z_ref)
        den_ref[...] = jnp.zeros_like(den_ref)

    base = pl.multiple_of(bases_ref[g], 16)
    dstr = dst_ref[0]
    hs16 = hs_ref[:, 0:H].astype(bf16)
    e16 = e_ref[...]

    idsW = lax.broadcasted_iota(i32, (W, T), 0) + base
    PT = (idsW == dstr).astype(bf16)

    cdn = (((0,), (0,)), ((), ()))
    h_win = h16_ref[pl.ds(base, W), :]
    hd16 = lax.dot_general(PT, h_win, cdn, preferred_element_type=f32).astype(bf16)

    cat = jnp.concatenate([hs16, e16, hd16], axis=1)
    f = jnp.dot(cat, A192_ref[...], preferred_element_type=f32)
    f = f + bA_ref[...]
    f = jnp.where(f >= 0, f, 0.2 * f)

    fa16 = (f * attn_ref[...]).astype(bf16)
    lg = jnp.dot(fa16, ones_ref[...], preferred_element_type=f32)
    ex = jnp.exp(lg)
    ex16 = ex.astype(bf16)

    den_ref[pl.ds(base, W), :] += jnp.dot(PT, ex16, preferred_element_type=f32)

    exb = jnp.dot(ex16, exp8_ref[...], preferred_element_type=f32).astype(bf16)
    hs4 = jnp.concatenate([hs16, hs16, hs16, hs16], axis=1)
    p_cat = hs4 * exb
    z_ref[pl.ds(base, W), :] += jnp.dot(PT, p_cat, preferred_element_type=f32)


def _edge(bases, hs_pad, e16, dst3, h16, A192l, bAl, attnl, ones_blk, exp8):
    grid_spec = pltpu.PrefetchScalarGridSpec(
        num_scalar_prefetch=1,
        grid=(G,),
        in_specs=[
            pl.BlockSpec((T, D), lambda g, b: (g, 0)),
            pl.BlockSpec((T, H), lambda g, b: (g, 0)),
            pl.BlockSpec((1, 1, T), lambda g, b: (g, 0, 0)),
            pl.BlockSpec((N, H), lambda g, b: (0, 0)),
            pl.BlockSpec((3 * H, 4 * H), lambda g, b: (0, 0)),
            pl.BlockSpec((1, 4 * H), lambda g, b: (0, 0)),
            pl.BlockSpec((1, 4 * H), lambda g, b: (0, 0)),
            pl.BlockSpec((4 * H, 8), lambda g, b: (0, 0)),
            pl.BlockSpec((8, 4 * H), lambda g, b: (0, 0)),
        ],
        out_specs=[
            pl.BlockSpec((N, 4 * H), lambda g, b: (0, 0)),
            pl.BlockSpec((N, 8), lambda g, b: (0, 0)),
        ],
    )
    return pl.pallas_call(
        _edge_body,
        grid_spec=grid_spec,
        out_shape=[jax.ShapeDtypeStruct((N, 4 * H), f32),
                   jax.ShapeDtypeStruct((N, 8), f32)],
    )(bases, hs_pad, e16, dst3, h16, A192l, bAl, attnl, ones_blk, exp8)


def _node_body(z_ref, den_ref, h_ref, WW_ref,
               bm_ref, h_out_ref, h16_out_ref):
    u = None
    for k in range(HEADS):
        r = 1.0 / (den_ref[:, k:k + 1] + 1e-9)
        q = z_ref[:, k * H:(k + 1) * H] * r
        t = jnp.dot(q, WW_ref[k], preferred_element_type=f32)
        u = t if u is None else u + t
    hn = jax.nn.gelu(u + bm_ref[...])
    h = hn + h_ref[:, 0:H]
    h_out_ref[:, 0:H] = h
    h_out_ref[:, H:D] = jnp.zeros((h.shape[0], D - H), f32)
    h16_out_ref[...] = h.astype(bf16)


def _node(z, den, h, WWl, bm_r):
    blk = 2000
    return pl.pallas_call(
        _node_body,
        grid=(N // blk,),
        in_specs=[
            pl.BlockSpec((blk, 4 * H), lambda b: (b, 0)),
            pl.BlockSpec((blk, 8), lambda b: (b, 0)),
            pl.BlockSpec((blk, D), lambda b: (b, 0)),
            pl.BlockSpec((HEADS, H, H), lambda b: (0, 0, 0)),
            pl.BlockSpec((1, H), lambda b: (0, 0)),
        ],
        out_specs=[
            pl.BlockSpec((blk, D), lambda b: (b, 0)),
            pl.BlockSpec((blk, H), lambda b: (b, 0)),
        ],
        out_shape=[
            jax.ShapeDtypeStruct((N, D), f32),
            jax.ShapeDtypeStruct((N, H), bf16),
        ],
    )(z, den, h, WWl, bm_r)


_NF = D + (LAYERS + 1) * H


def _final_body(x_ref, h0_ref, h1_ref, h2_ref, h3_ref, h4_ref, out_ref,
                gmax_ref):
    p = pl.program_id(0)
    b = pl.program_id(1)
    hs = (h0_ref, h1_ref, h2_ref, h3_ref, h4_ref)

    @pl.when((p == 0) & (b == 0))
    def _():
        gmax_ref[...] = jnp.full_like(gmax_ref, -jnp.inf)

    @pl.when(p == 0)
    def _():
        m = jnp.max(x_ref[...], axis=0, keepdims=True)
        gmax_ref[0:1, 0:D] = jnp.maximum(gmax_ref[0:1, 0:D], m)
        for i, hr in enumerate(hs):
            lo = D + i * H
            m = jnp.max(hr[:, 0:H], axis=0, keepdims=True)
            gmax_ref[0:1, lo:lo + H] = jnp.maximum(gmax_ref[0:1, lo:lo + H], m)

    @pl.when(p == 1)
    def _():
        out_ref[:, 0:D] = x_ref[...]
        for i, hr in enumerate(hs):
            lo = D + i * H
            out_ref[:, lo:lo + H] = hr[:, 0:H]
        g = gmax_ref[0:1, 0:_NF]
        out_ref[:, _NF:2 * _NF] = jnp.broadcast_to(g, (x_ref.shape[0], _NF))


def _final(x, h0, h1, h2, h3, h4):
    blk = 2000
    return pl.pallas_call(
        _final_body,
        grid=(2, N // blk),
        in_specs=[pl.BlockSpec((blk, D), lambda p, b: (b, 0))] * 6,
        out_specs=pl.BlockSpec((blk, 2 * _NF), lambda p, b: (b, 0)),
        out_shape=jax.ShapeDtypeStruct((N, 2 * _NF), f32),
        scratch_shapes=[pltpu.VMEM((8, _NF), f32)],
    )(x, h0, h1, h2, h3, h4)


def kernel(x, edge_index, edge_type, rc_att, W_feat, b_feat, ET, Wrc, brc, A,
           bA, attn, Wn, Wm, bm):
    src = edge_index[0]
    dst = edge_index[1]

    dst_s, perm = lax.sort_key_val(dst, jnp.arange(E, dtype=i32))
    src_s = jnp.take(src, perm)
    et_s = jnp.take(edge_type, perm)
    rc_s = jnp.take(rc_att, perm, axis=0)

    bases = jnp.minimum((dst_s[::T] // 16) * 16, N - W).astype(i32)

    et3 = et_s.reshape(G, 1, T)
    dst3 = dst_s.reshape(G, 1, T)
    rc03 = rc_s[:, 0].reshape(G, 1, T)
    rc13 = rc_s[:, 1].reshape(G, 1, T)
    src_pad = jnp.concatenate([src_s, jnp.zeros(E_PAD - E, i32)])
    idx2d = src_pad.reshape(E_PAD // CHUNK, CHUNK)

    ET48 = jnp.concatenate([ET, jnp.zeros((NETP - NET, H), f32)]).astype(bf16)
    Wrc8 = jnp.concatenate([Wrc, jnp.zeros((6, H), f32)]).astype(bf16)
    brc_r = brc.reshape(1, H)
    A192 = A.astype(bf16)
    bA_r = bA.reshape(LAYERS, 1, HEADS * H)
    attn_r = attn.reshape(LAYERS, 1, HEADS * H)
    col = jnp.arange(HEADS * H) // H
    ones_blk = (col[:, None] == jnp.arange(8)[None, :]).astype(bf16)
    exp8 = (jnp.arange(8)[:, None] == col[None, :]).astype(bf16)
    Wn4 = Wn.reshape(LAYERS, H, HEADS, H).transpose(0, 2, 1, 3)
    Wm4 = Wm.reshape(LAYERS, HEADS, H, H)
    WW = jnp.einsum("lkab,lkbc->lkac", Wn4, Wm4)
    bm_r = bm.reshape(LAYERS, 1, H)

    h, h16 = _prolog(x, W_feat, b_feat)
    e16 = _embed(et3, rc03, rc13, ET48, Wrc8, brc_r)
    feats = [h]

    for l in range(LAYERS):
        hs_pad = _sc_gather(h, idx2d)
        z, den = _edge(bases, hs_pad, e16, dst3, h16,
                       A192[l], bA_r[l], attn_r[l], ones_blk, exp8)
        h, h16 = _node(z, den, h, WW[l], bm_r[l])
        feats.append(h)

    return _final(x, *feats)


# speedup vs baseline: 40.6630x; 1.0801x over previous
import functools

import jax
import jax.numpy as jnp
from jax import lax
from jax.experimental import pallas as pl
from jax.experimental.pallas import tpu as pltpu
from jax.experimental.pallas import tpu_sc as plsc

N = 10000
E = 320000
D = 128
H = 64
HEADS = 4
LAYERS = 4
NET = 41
NETP = 48

T = 2560
G = E // T
W = 128

NW = 32
CHUNK = 128
CPW = 80
E_PAD = NW * CPW * CHUNK
NBUF = 6

f32 = jnp.float32
bf16 = jnp.bfloat16
i32 = jnp.int32


K_INFLIGHT = 3
SC_TOT = 84


def _sc_gather_body(h16_hbm, idx_hbm, out_hbm, idx_v, rows_v, gsem, wsem):
    cid = lax.axis_index("c")
    sid = lax.axis_index("s")
    wid = sid * 2 + cid
    pltpu.sync_copy(idx_hbm.at[pl.ds(wid * CPW, CPW)], idx_v)

    def start_g(j, b):
        pltpu.async_copy(h16_hbm.at[idx_v.at[j]], rows_v.at[b], gsem.at[b])

    def wait_g(j, b):
        pltpu.make_async_copy(
            h16_hbm.at[idx_v.at[j]], rows_v.at[b], gsem.at[b]
        ).wait()

    def start_w(j, b):
        c = wid * CPW + j
        pltpu.async_copy(rows_v.at[b], out_hbm.at[pl.ds(c * CHUNK, CHUNK)],
                         wsem.at[b])

    def wait_w(j, b):
        c = wid * CPW + j
        pltpu.make_async_copy(
            rows_v.at[b], out_hbm.at[pl.ds(c * CHUNK, CHUNK)], wsem.at[b]
        ).wait()

    @pl.loop(0, SC_TOT, step=NBUF)
    def _(ii):
        for db in range(NBUF):
            i = ii + db
            b = db
            @pl.when((i >= NBUF) & (i < CPW))
            def _():
                wait_w(i - NBUF, b)

            @pl.when(i < CPW)
            def _():
                start_g(i, b)

            j = i - K_INFLIGHT
            bj = (db - K_INFLIGHT) % NBUF

            @pl.when((j >= 0) & (j < CPW))
            def _():
                wait_g(j, bj)
                start_w(j, bj)

    for d in range(NBUF):
        j = CPW - NBUF + d
        wait_w(j, j % NBUF)


def _sc_gather(h_pad, idx2d):
    mesh = plsc.VectorSubcoreMesh(core_axis_name="c", subcore_axis_name="s")
    kern = pl.kernel(
        _sc_gather_body,
        out_type=jax.ShapeDtypeStruct((E_PAD, D), f32),
        mesh=mesh,
        scratch_types=[
            pltpu.VMEM((CPW, CHUNK), i32),
            pltpu.VMEM((NBUF, CHUNK, D), f32),
            pltpu.SemaphoreType.DMA((NBUF,)),
            pltpu.SemaphoreType.DMA((NBUF,)),
        ],
    )
    return kern(h_pad, idx2d)


def _prolog_body(x_ref, wf_ref, bf_ref, h_ref, h16_ref):
    h = jnp.dot(x_ref[...], wf_ref[...], preferred_element_type=f32)
    h = jax.nn.gelu(h + bf_ref[...])
    h_ref[:, 0:H] = h
    h_ref[:, H:D] = jnp.zeros((h.shape[0], D - H), f32)
    h16_ref[...] = h.astype(bf16)


def _prolog(x, W_feat, b_feat):
    blk = 2000
    return pl.pallas_call(
        _prolog_body,
        grid=(N // blk,),
        in_specs=[
            pl.BlockSpec((blk, D), lambda b: (b, 0)),
            pl.BlockSpec((D, H), lambda b: (0, 0)),
            pl.BlockSpec((1, H), lambda b: (0, 0)),
        ],
        out_specs=[
            pl.BlockSpec((blk, D), lambda b: (b, 0)),
            pl.BlockSpec((blk, H), lambda b: (b, 0)),
        ],
        out_shape=[
            jax.ShapeDtypeStruct((N, D), f32),
            jax.ShapeDtypeStruct((N, H), bf16),
        ],
    )(x, W_feat, b_feat.reshape(1, H))


def _embed_body(et_ref, rc0_ref, rc1_ref, ET_ref, Wrc_ref, brc_ref, e_ref):
    etr = et_ref[0]
    ids48 = lax.broadcasted_iota(i32, (NETP, T), 0)
    OH = (ids48 == etr).astype(bf16)
    rc0 = rc0_ref[0].astype(bf16)
    rc1 = rc1_ref[0].astype(bf16)
    RC = jnp.concatenate([rc0, rc1, jnp.zeros((6, T), bf16)], axis=0)
    cdn = (((0,), (0,)), ((), ()))
    e_f = lax.dot_general(OH, ET_ref[...], cdn, preferred_element_type=f32)
    e_f = e_f + lax.dot_general(RC, Wrc_ref[...], cdn, preferred_element_type=f32)
    e_ref[...] = (e_f + brc_ref[...]).astype(bf16)


def _embed(et3, rc03, rc13, ET48, Wrc8, brc_r):
    return pl.pallas_call(
        _embed_body,
        grid=(G,),
        in_specs=[
            pl.BlockSpec((1, 1, T), lambda g: (g, 0, 0)),
            pl.BlockSpec((1, 1, T), lambda g: (g, 0, 0)),
            pl.BlockSpec((1, 1, T), lambda g: (g, 0, 0)),
            pl.BlockSpec((NETP, H), lambda g: (0, 0)),
            pl.BlockSpec((8, H), lambda g: (0, 0)),
            pl.BlockSpec((1, H), lambda g: (0, 0)),
        ],
        out_specs=pl.BlockSpec((T, H), lambda g: (g, 0)),
        out_shape=jax.ShapeDtypeStruct((E, H), bf16),
    )(et3, rc03, rc13, ET48, Wrc8, brc_r)


def _edge_body(bases_ref, hs_ref, e_ref, dst_ref, h16_ref,
               A192_ref, bA_ref, attn_ref, ones_ref, exp8_ref,
               z_ref, den_ref):
    g = pl.program_id(0)

    @pl.when(g == 0)
    def _():
        z_ref[...] = jnp.zeros_like(z_ref)
        den_ref[...] = jnp.zeros_like(den_ref)

    base = pl.multiple_of(bases_ref[g], 16)
    dstr = dst_ref[0]
    hs16 = hs_ref[:, 0:H].astype(bf16)
    e16 = e_ref[...]

    idsW = lax.broadcasted_iota(i32, (W, T), 0) + base
    PT = (idsW == dstr).astype(bf16)

    cdn = (((0,), (0,)), ((), ()))
    h_win = h16_ref[pl.ds(base, W), :]
    hd16 = lax.dot_general(PT, h_win, cdn, preferred_element_type=f32).astype(bf16)

    cat = jnp.concatenate([hs16, e16, hd16], axis=1)
    f = jnp.dot(cat, A192_ref[...], preferred_element_type=f32)
    f = f + bA_ref[...]
    f = jnp.where(f >= 0, f, 0.2 * f)

    fa16 = (f * attn_ref[...]).astype(bf16)
    lg = jnp.dot(fa16, ones_ref[...], preferred_element_type=f32)
    ex = jnp.exp(lg)
    ex16 = ex.astype(bf16)

    den_ref[pl.ds(base, W), :] += jnp.dot(PT, ex16, preferred_element_type=f32)

    exb = jnp.dot(ex16, exp8_ref[...], preferred_element_type=f32).astype(bf16)
    hs4 = jnp.concatenate([hs16, hs16, hs16, hs16], axis=1)
    p_cat = hs4 * exb
    z_ref[pl.ds(base, W), :] += jnp.dot(PT, p_cat, preferred_element_type=f32)


def _edge(bases, hs_pad, e16, dst3, h16, A192l, bAl, attnl, ones_blk, exp8):
    grid_spec = pltpu.PrefetchScalarGridSpec(
        num_scalar_prefetch=1,
        grid=(G,),
        in_specs=[
            pl.BlockSpec((T, D), lambda g, b: (g, 0)),
            pl.BlockSpec((T, H), lambda g, b: (g, 0)),
            pl.BlockSpec((1, 1, T), lambda g, b: (g, 0, 0)),
            pl.BlockSpec((N, H), lambda g, b: (0, 0)),
            pl.BlockSpec((3 * H, 4 * H), lambda g, b: (0, 0)),
            pl.BlockSpec((1, 4 * H), lambda g, b: (0, 0)),
            pl.BlockSpec((1, 4 * H), lambda g, b: (0, 0)),
            pl.BlockSpec((4 * H, 8), lambda g, b: (0, 0)),
            pl.BlockSpec((8, 4 * H), lambda g, b: (0, 0)),
        ],
        out_specs=[
            pl.BlockSpec((N, 4 * H), lambda g, b: (0, 0)),
            pl.BlockSpec((N, 8), lambda g, b: (0, 0)),
        ],
    )
    return pl.pallas_call(
        _edge_body,
        grid_spec=grid_spec,
        out_shape=[jax.ShapeDtypeStruct((N, 4 * H), f32),
                   jax.ShapeDtypeStruct((N, 8), f32)],
    )(bases, hs_pad, e16, dst3, h16, A192l, bAl, attnl, ones_blk, exp8)


def _node_body(z_ref, den_ref, h_ref, WW_ref,
               bm_ref, h_out_ref, h16_out_ref):
    u = None
    for k in range(HEADS):
        r = 1.0 / (den_ref[:, k:k + 1] + 1e-9)
        q = z_ref[:, k * H:(k + 1) * H] * r
        t = jnp.dot(q, WW_ref[k], preferred_element_type=f32)
        u = t if u is None else u + t
    hn = jax.nn.gelu(u + bm_ref[...])
    h = hn + h_ref[:, 0:H]
    h_out_ref[:, 0:H] = h
    h_out_ref[:, H:D] = jnp.zeros((h.shape[0], D - H), f32)
    h16_out_ref[...] = h.astype(bf16)


def _node(z, den, h, WWl, bm_r):
    blk = 2000
    return pl.pallas_call(
        _node_body,
        grid=(N // blk,),
        in_specs=[
            pl.BlockSpec((blk, 4 * H), lambda b: (b, 0)),
            pl.BlockSpec((blk, 8), lambda b: (b, 0)),
            pl.BlockSpec((blk, D), lambda b: (b, 0)),
            pl.BlockSpec((HEADS, H, H), lambda b: (0, 0, 0)),
            pl.BlockSpec((1, H), lambda b: (0, 0)),
        ],
        out_specs=[
            pl.BlockSpec((blk, D), lambda b: (b, 0)),
            pl.BlockSpec((blk, H), lambda b: (b, 0)),
        ],
        out_shape=[
            jax.ShapeDtypeStruct((N, D), f32),
            jax.ShapeDtypeStruct((N, H), bf16),
        ],
    )(z, den, h, WWl, bm_r)


_NF = D + (LAYERS + 1) * H


def _final_body(x_ref, h0_ref, h1_ref, h2_ref, h3_ref, h4_ref, out_ref,
                gmax_ref):
    p = pl.program_id(0)
    b = pl.program_id(1)
    hs = (h0_ref, h1_ref, h2_ref, h3_ref, h4_ref)

    @pl.when((p == 0) & (b == 0))
    def _():
        gmax_ref[...] = jnp.full_like(gmax_ref, -jnp.inf)

    @pl.when(p == 0)
    def _():
        m = jnp.max(x_ref[...], axis=0, keepdims=True)
        gmax_ref[0:1, 0:D] = jnp.maximum(gmax_ref[0:1, 0:D], m)
        for i, hr in enumerate(hs):
            lo = D + i * H
            m = jnp.max(hr[:, 0:H], axis=0, keepdims=True)
            gmax_ref[0:1, lo:lo + H] = jnp.maximum(gmax_ref[0:1, lo:lo + H], m)

    @pl.when(p == 1)
    def _():
        out_ref[:, 0:D] = x_ref[...]
        for i, hr in enumerate(hs):
            lo = D + i * H
            out_ref[:, lo:lo + H] = hr[:, 0:H]
        g = gmax_ref[0:1, 0:_NF]
        out_ref[:, _NF:2 * _NF] = jnp.broadcast_to(g, (x_ref.shape[0], _NF))


def _final(x, h0, h1, h2, h3, h4):
    blk = 2000
    return pl.pallas_call(
        _final_body,
        grid=(2, N // blk),
        in_specs=[pl.BlockSpec((blk, D), lambda p, b: (b, 0))] * 6,
        out_specs=pl.BlockSpec((blk, 2 * _NF), lambda p, b: (b, 0)),
        out_shape=jax.ShapeDtypeStruct((N, 2 * _NF), f32),
        scratch_shapes=[pltpu.VMEM((8, _NF), f32)],
    )(x, h0, h1, h2, h3, h4)


def kernel(x, edge_index, edge_type, rc_att, W_feat, b_feat, ET, Wrc, brc, A,
           bA, attn, Wn, Wm, bm):
    src = edge_index[0]
    dst = edge_index[1]

    dst_s, perm = lax.sort_key_val(dst, jnp.arange(E, dtype=i32))
    src_s = jnp.take(src, perm)
    et_s = jnp.take(edge_type, perm)
    rc_s = jnp.take(rc_att, perm, axis=0)

    bases = jnp.minimum((dst_s[::T] // 16) * 16, N - W).astype(i32)

    et3 = et_s.reshape(G, 1, T)
    dst3 = dst_s.reshape(G, 1, T)
    rc03 = rc_s[:, 0].reshape(G, 1, T)
    rc13 = rc_s[:, 1].reshape(G, 1, T)
    src_pad = jnp.concatenate([src_s, jnp.zeros(E_PAD - E, i32)])
    idx2d = src_pad.reshape(E_PAD // CHUNK, CHUNK)

    ET48 = jnp.concatenate([ET, jnp.zeros((NETP - NET, H), f32)]).astype(bf16)
    Wrc8 = jnp.concatenate([Wrc, jnp.zeros((6, H), f32)]).astype(bf16)
    brc_r = brc.reshape(1, H)
    A192 = A.astype(bf16)
    bA_r = bA.reshape(LAYERS, 1, HEADS * H)
    attn_r = attn.reshape(LAYERS, 1, HEADS * H)
    col = jnp.arange(HEADS * H) // H
    ones_blk = (col[:, None] == jnp.arange(8)[None, :]).astype(bf16)
    exp8 = (jnp.arange(8)[:, None] == col[None, :]).astype(bf16)
    Wn4 = Wn.reshape(LAYERS, H, HEADS, H).transpose(0, 2, 1, 3)
    Wm4 = Wm.reshape(LAYERS, HEADS, H, H)
    WW = jnp.einsum("lkab,lkbc->lkac", Wn4, Wm4)
    bm_r = bm.reshape(LAYERS, 1, H)

    h, h16 = _prolog(x, W_feat, b_feat)
    e16 = _embed(et3, rc03, rc13, ET48, Wrc8, brc_r)
    feats = [h]

    for l in range(LAYERS):
        hs_pad = _sc_gather(h, idx2d)
        z, den = _edge(bases, hs_pad, e16, dst3, h16,
                       A192[l], bA_r[l], attn_r[l], ones_blk, exp8)
        h, h16 = _node(z, den, h, WW[l], bm_r[l])
        feats.append(h)

    return _final(x, *feats)


# speedup vs baseline: 47.1486x; 1.1595x over previous
import functools

import jax
import jax.numpy as jnp
from jax import lax
from jax.experimental import pallas as pl
from jax.experimental.pallas import tpu as pltpu
from jax.experimental.pallas import tpu_sc as plsc

N = 10000
E = 320000
D = 128
H = 64
HEADS = 4
LAYERS = 4
NET = 41
NETP = 48

T = 2560
G = E // T
W = 128

NW = 32
CHUNK = 128
CPW = 80
E_PAD = NW * CPW * CHUNK
NBUF = 6

f32 = jnp.float32
bf16 = jnp.bfloat16
i32 = jnp.int32


K_INFLIGHT = 3
SC_TOT = 84


def _sc_gather_body(chunk_lo, cpw, tot,
                    h16_hbm, idx_hbm, out_hbm, idx_v, rows_v, gsem, wsem):
    cid = lax.axis_index("c")
    sid = lax.axis_index("s")
    wid = sid * 2 + cid
    pltpu.sync_copy(idx_hbm.at[pl.ds(chunk_lo + wid * cpw, cpw)], idx_v)

    def start_g(j, b):
        pltpu.async_copy(h16_hbm.at[idx_v.at[j]], rows_v.at[b], gsem.at[b])

    def wait_g(j, b):
        pltpu.make_async_copy(
            h16_hbm.at[idx_v.at[j]], rows_v.at[b], gsem.at[b]
        ).wait()

    def start_w(j, b):
        c = wid * cpw + j
        pltpu.async_copy(rows_v.at[b], out_hbm.at[pl.ds(c * CHUNK, CHUNK)],
                         wsem.at[b])

    def wait_w(j, b):
        c = wid * cpw + j
        pltpu.make_async_copy(
            rows_v.at[b], out_hbm.at[pl.ds(c * CHUNK, CHUNK)], wsem.at[b]
        ).wait()

    @pl.loop(0, tot, step=NBUF)
    def _(ii):
        for db in range(NBUF):
            i = ii + db
            b = db
            @pl.when((i >= NBUF) & (i < cpw))
            def _():
                wait_w(i - NBUF, b)

            @pl.when(i < cpw)
            def _():
                start_g(i, b)

            j = i - K_INFLIGHT
            bj = (db - K_INFLIGHT) % NBUF

            @pl.when((j >= 0) & (j < cpw))
            def _():
                wait_g(j, bj)
                start_w(j, bj)

    for d in range(NBUF):
        j = cpw - NBUF + d
        wait_w(j, j % NBUF)


SPLIT_ROWS = E_PAD // 2
CPW_S = 40


def _sc_gather(h_pad, idx2d, phase):
    mesh = plsc.VectorSubcoreMesh(core_axis_name="c", subcore_axis_name="s")
    kern = pl.kernel(
        functools.partial(_sc_gather_body, phase * (SPLIT_ROWS // CHUNK),
                          CPW_S, 48),
        out_type=jax.ShapeDtypeStruct((SPLIT_ROWS, D), f32),
        mesh=mesh,
        scratch_types=[
            pltpu.VMEM((CPW_S, CHUNK), i32),
            pltpu.VMEM((NBUF, CHUNK, D), f32),
            pltpu.SemaphoreType.DMA((NBUF,)),
            pltpu.SemaphoreType.DMA((NBUF,)),
        ],
    )
    return kern(h_pad, idx2d)


def _prolog_body(x_ref, wf_ref, bf_ref, h_ref, h16_ref):
    h = jnp.dot(x_ref[...], wf_ref[...], preferred_element_type=f32)
    h = jax.nn.gelu(h + bf_ref[...])
    h_ref[:, 0:H] = h
    h_ref[:, H:D] = jnp.zeros((h.shape[0], D - H), f32)
    h16_ref[...] = h.astype(bf16)


def _prolog(x, W_feat, b_feat):
    blk = 2000
    return pl.pallas_call(
        _prolog_body,
        grid=(N // blk,),
        in_specs=[
            pl.BlockSpec((blk, D), lambda b: (b, 0)),
            pl.BlockSpec((D, H), lambda b: (0, 0)),
            pl.BlockSpec((1, H), lambda b: (0, 0)),
        ],
        out_specs=[
            pl.BlockSpec((blk, D), lambda b: (b, 0)),
            pl.BlockSpec((blk, H), lambda b: (b, 0)),
        ],
        out_shape=[
            jax.ShapeDtypeStruct((N, D), f32),
            jax.ShapeDtypeStruct((N, H), bf16),
        ],
    )(x, W_feat, b_feat.reshape(1, H))


def _embed_body(et_ref, rc0_ref, rc1_ref, ET_ref, Wrc_ref, brc_ref, e_ref):
    etr = et_ref[0]
    ids48 = lax.broadcasted_iota(i32, (NETP, T), 0)
    OH = (ids48 == etr).astype(bf16)
    rc0 = rc0_ref[0].astype(bf16)
    rc1 = rc1_ref[0].astype(bf16)
    RC = jnp.concatenate([rc0, rc1, jnp.zeros((6, T), bf16)], axis=0)
    cdn = (((0,), (0,)), ((), ()))
    e_f = lax.dot_general(OH, ET_ref[...], cdn, preferred_element_type=f32)
    e_f = e_f + lax.dot_general(RC, Wrc_ref[...], cdn, preferred_element_type=f32)
    e_ref[...] = (e_f + brc_ref[...]).astype(bf16)


def _embed(et3, rc03, rc13, ET48, Wrc8, brc_r):
    return pl.pallas_call(
        _embed_body,
        grid=(G,),
        in_specs=[
            pl.BlockSpec((1, 1, T), lambda g: (g, 0, 0)),
            pl.BlockSpec((1, 1, T), lambda g: (g, 0, 0)),
            pl.BlockSpec((1, 1, T), lambda g: (g, 0, 0)),
            pl.BlockSpec((NETP, H), lambda g: (0, 0)),
            pl.BlockSpec((8, H), lambda g: (0, 0)),
            pl.BlockSpec((1, H), lambda g: (0, 0)),
        ],
        out_specs=pl.BlockSpec((T, H), lambda g: (g, 0)),
        out_shape=jax.ShapeDtypeStruct((E, H), bf16),
    )(et3, rc03, rc13, ET48, Wrc8, brc_r)


def _edge_body(bases_ref, hs_ref, e_ref, dst_ref, h16_ref,
               A192_ref, bA_ref, attn_ref, ones_ref, exp8_ref,
               z_in_ref, den_in_ref, z_ref, den_ref):
    g = pl.program_id(0)

    @pl.when(g == 0)
    def _():
        z_ref[...] = z_in_ref[...]
        den_ref[...] = den_in_ref[...]

    base = pl.multiple_of(bases_ref[g], 16)
    dstr = dst_ref[0]
    hs16 = hs_ref[:, 0:H].astype(bf16)
    e16 = e_ref[...]

    idsW = lax.broadcasted_iota(i32, (W, T), 0) + base
    PT = (idsW == dstr).astype(bf16)

    cdn = (((0,), (0,)), ((), ()))
    h_win = h16_ref[pl.ds(base, W), :]
    hd16 = lax.dot_general(PT, h_win, cdn, preferred_element_type=f32).astype(bf16)

    cat = jnp.concatenate([hs16, e16, hd16], axis=1)
    f = jnp.dot(cat, A192_ref[...], preferred_element_type=f32)
    f = f + bA_ref[...]
    f = jnp.where(f >= 0, f, 0.2 * f)

    fa16 = (f * attn_ref[...]).astype(bf16)
    lg = jnp.dot(fa16, ones_ref[...], preferred_element_type=f32)
    ex = jnp.exp(lg)
    ex16 = ex.astype(bf16)

    den_ref[pl.ds(base, W), :] += jnp.dot(PT, ex16, preferred_element_type=f32)

    exb = jnp.dot(ex16, exp8_ref[...], preferred_element_type=f32).astype(bf16)
    hs4 = jnp.concatenate([hs16, hs16, hs16, hs16], axis=1)
    p_cat = hs4 * exb
    z_ref[pl.ds(base, W), :] += jnp.dot(PT, p_cat, preferred_element_type=f32)


def _edge(bases_ph, hs_ph, e16, dst3, h16, A192l, bAl, attnl, ones_blk, exp8,
          z_in, den_in, tile_off, ntiles):
    grid_spec = pltpu.PrefetchScalarGridSpec(
        num_scalar_prefetch=1,
        grid=(ntiles,),
        in_specs=[
            pl.BlockSpec((T, D), lambda g, b: (g, 0)),
            pl.BlockSpec((T, H), lambda g, b: (g + tile_off, 0)),
            pl.BlockSpec((1, 1, T), lambda g, b: (g + tile_off, 0, 0)),
            pl.BlockSpec((N, H), lambda g, b: (0, 0)),
            pl.BlockSpec((3 * H, 4 * H), lambda g, b: (0, 0)),
            pl.BlockSpec((1, 4 * H), lambda g, b: (0, 0)),
            pl.BlockSpec((1, 4 * H), lambda g, b: (0, 0)),
            pl.BlockSpec((4 * H, 8), lambda g, b: (0, 0)),
            pl.BlockSpec((8, 4 * H), lambda g, b: (0, 0)),
            pl.BlockSpec((N, 4 * H), lambda g, b: (0, 0)),
            pl.BlockSpec((N, 8), lambda g, b: (0, 0)),
        ],
        out_specs=[
            pl.BlockSpec((N, 4 * H), lambda g, b: (0, 0)),
            pl.BlockSpec((N, 8), lambda g, b: (0, 0)),
        ],
    )
    return pl.pallas_call(
        _edge_body,
        grid_spec=grid_spec,
        out_shape=[jax.ShapeDtypeStruct((N, 4 * H), f32),
                   jax.ShapeDtypeStruct((N, 8), f32)],
    )(bases_ph, hs_ph, e16, dst3, h16, A192l, bAl, attnl, ones_blk, exp8,
      z_in, den_in)


def _node_body(z_ref, den_ref, h_ref, WW_ref,
               bm_ref, h_out_ref, h16_out_ref):
    u = None
    for k in range(HEADS):
        r = 1.0 / (den_ref[:, k:k + 1] + 1e-9)
        q = z_ref[:, k * H:(k + 1) * H] * r
        t = jnp.dot(q, WW_ref[k], preferred_element_type=f32)
        u = t if u is None else u + t
    hn = jax.nn.gelu(u + bm_ref[...])
    h = hn + h_ref[:, 0:H]
    h_out_ref[:, 0:H] = h
    h_out_ref[:, H:D] = jnp.zeros((h.shape[0], D - H), f32)
    h16_out_ref[...] = h.astype(bf16)


def _node(z, den, h, WWl, bm_r):
    blk = 2000
    return pl.pallas_call(
        _node_body,
        grid=(N // blk,),
        in_specs=[
            pl.BlockSpec((blk, 4 * H), lambda b: (b, 0)),
            pl.BlockSpec((blk, 8), lambda b: (b, 0)),
            pl.BlockSpec((blk, D), lambda b: (b, 0)),
            pl.BlockSpec((HEADS, H, H), lambda b: (0, 0, 0)),
            pl.BlockSpec((1, H), lambda b: (0, 0)),
        ],
        out_specs=[
            pl.BlockSpec((blk, D), lambda b: (b, 0)),
            pl.BlockSpec((blk, H), lambda b: (b, 0)),
        ],
        out_shape=[
            jax.ShapeDtypeStruct((N, D), f32),
            jax.ShapeDtypeStruct((N, H), bf16),
        ],
    )(z, den, h, WWl, bm_r)


_NF = D + (LAYERS + 1) * H


def _final_body(x_ref, h0_ref, h1_ref, h2_ref, h3_ref, h4_ref, out_ref,
                gmax_ref):
    p = pl.program_id(0)
    b = pl.program_id(1)
    hs = (h0_ref, h1_ref, h2_ref, h3_ref, h4_ref)

    @pl.when((p == 0) & (b == 0))
    def _():
        gmax_ref[...] = jnp.full_like(gmax_ref, -jnp.inf)

    @pl.when(p == 0)
    def _():
        m = jnp.max(x_ref[...], axis=0, keepdims=True)
        gmax_ref[0:1, 0:D] = jnp.maximum(gmax_ref[0:1, 0:D], m)
        for i, hr in enumerate(hs):
            lo = D + i * H
            m = jnp.max(hr[:, 0:H], axis=0, keepdims=True)
            gmax_ref[0:1, lo:lo + H] = jnp.maximum(gmax_ref[0:1, lo:lo + H], m)

    @pl.when(p == 1)
    def _():
        out_ref[:, 0:D] = x_ref[...]
        for i, hr in enumerate(hs):
            lo = D + i * H
            out_ref[:, lo:lo + H] = hr[:, 0:H]
        g = gmax_ref[0:1, 0:_NF]
        out_ref[:, _NF:2 * _NF] = jnp.broadcast_to(g, (x_ref.shape[0], _NF))


def _final(x, h0, h1, h2, h3, h4):
    blk = 2000
    return pl.pallas_call(
        _final_body,
        grid=(2, N // blk),
        in_specs=[pl.BlockSpec((blk, D), lambda p, b: (b, 0))] * 6,
        out_specs=pl.BlockSpec((blk, 2 * _NF), lambda p, b: (b, 0)),
        out_shape=jax.ShapeDtypeStruct((N, 2 * _NF), f32),
        scratch_shapes=[pltpu.VMEM((8, _NF), f32)],
    )(x, h0, h1, h2, h3, h4)


def kernel(x, edge_index, edge_type, rc_att, W_feat, b_feat, ET, Wrc, brc, A,
           bA, attn, Wn, Wm, bm):
    src = edge_index[0]
    dst = edge_index[1]

    dst_s, perm = lax.sort_key_val(dst, jnp.arange(E, dtype=i32))
    src_s = jnp.take(src, perm)
    et_s = jnp.take(edge_type, perm)
    rc_s = jnp.take(rc_att, perm, axis=0)

    bases = jnp.minimum((dst_s[::T] // 16) * 16, N - W).astype(i32)

    et3 = et_s.reshape(G, 1, T)
    dst3 = dst_s.reshape(G, 1, T)
    rc03 = rc_s[:, 0].reshape(G, 1, T)
    rc13 = rc_s[:, 1].reshape(G, 1, T)
    src_pad = jnp.concatenate([src_s, jnp.zeros(E_PAD - E, i32)])
    idx2d = src_pad.reshape(E_PAD // CHUNK, CHUNK)

    ET48 = jnp.concatenate([ET, jnp.zeros((NETP - NET, H), f32)]).astype(bf16)
    Wrc8 = jnp.concatenate([Wrc, jnp.zeros((6, H), f32)]).astype(bf16)
    brc_r = brc.reshape(1, H)
    A192 = A.astype(bf16)
    bA_r = bA.reshape(LAYERS, 1, HEADS * H)
    attn_r = attn.reshape(LAYERS, 1, HEADS * H)
    col = jnp.arange(HEADS * H) // H
    ones_blk = (col[:, None] == jnp.arange(8)[None, :]).astype(bf16)
    exp8 = (jnp.arange(8)[:, None] == col[None, :]).astype(bf16)
    Wn4 = Wn.reshape(LAYERS, H, HEADS, H).transpose(0, 2, 1, 3)
    Wm4 = Wm.reshape(LAYERS, HEADS, H, H)
    WW = jnp.einsum("lkab,lkbc->lkac", Wn4, Wm4)
    bm_r = bm.reshape(LAYERS, 1, H)

    h, h16 = _prolog(x, W_feat, b_feat)
    e16 = _embed(et3, rc03, rc13, ET48, Wrc8, brc_r)
    feats = [h]

    GT0 = SPLIT_ROWS // T
    z0 = jnp.zeros((N, HEADS * H), f32)
    d0 = jnp.zeros((N, 8), f32)
    for l in range(LAYERS):
        hs0 = _sc_gather(h, idx2d, 0)
        hs1 = _sc_gather(h, idx2d, 1)
        z, den = _edge(bases[:GT0], hs0, e16, dst3, h16,
                       A192[l], bA_r[l], attn_r[l], ones_blk, exp8,
                       z0, d0, 0, GT0)
        z, den = _edge(bases[GT0:], hs1, e16, dst3, h16,
                       A192[l], bA_r[l], attn_r[l], ones_blk, exp8,
                       z, den, GT0, G - GT0)
        h, h16 = _node(z, den, h, WW[l], bm_r[l])
        feats.append(h)

    return _final(x, *feats)


# speedup vs baseline: 48.5971x; 1.0307x over previous
import functools

import jax
import jax.numpy as jnp
from jax import lax
from jax.experimental import pallas as pl
from jax.experimental.pallas import tpu as pltpu
from jax.experimental.pallas import tpu_sc as plsc

N = 10000
E = 320000
D = 128
H = 64
HEADS = 4
LAYERS = 4
NET = 41
NETP = 48

T = 2560
G = E // T
W = 128

NW = 32
CHUNK = 128
CPW = 80
E_PAD = NW * CPW * CHUNK
NBUF = 6

f32 = jnp.float32
bf16 = jnp.bfloat16
i32 = jnp.int32


K_INFLIGHT = 3
SC_TOT = 84


def _sc_gather_body(chunk_lo, cpw, tot,
                    h16_hbm, idx_hbm, out_hbm, idx_v, rows_v, gsem, wsem):
    cid = lax.axis_index("c")
    sid = lax.axis_index("s")
    wid = sid * 2 + cid
    pltpu.sync_copy(idx_hbm.at[chunk_lo // cpw + wid], idx_v)

    def start_g(j, b):
        pltpu.async_copy(h16_hbm.at[idx_v.at[j]], rows_v.at[b], gsem.at[b])

    def wait_g(j, b):
        pltpu.make_async_copy(
            h16_hbm.at[idx_v.at[j]], rows_v.at[b], gsem.at[b]
        ).wait()

    def start_w(j, b):
        c = wid * cpw + j
        pltpu.async_copy(rows_v.at[b], out_hbm.at[pl.ds(c * CHUNK, CHUNK)],
                         wsem.at[b])

    def wait_w(j, b):
        c = wid * cpw + j
        pltpu.make_async_copy(
            rows_v.at[b], out_hbm.at[pl.ds(c * CHUNK, CHUNK)], wsem.at[b]
        ).wait()

    @pl.loop(0, tot, step=NBUF)
    def _(ii):
        for db in range(NBUF):
            i = ii + db
            b = db
            @pl.when((i >= NBUF) & (i < cpw))
            def _():
                wait_w(i - NBUF, b)

            @pl.when(i < cpw)
            def _():
                start_g(i, b)

            j = i - K_INFLIGHT
            bj = (db - K_INFLIGHT) % NBUF

            @pl.when((j >= 0) & (j < cpw))
            def _():
                wait_g(j, bj)
                start_w(j, bj)

    for d in range(NBUF):
        j = cpw - NBUF + d
        wait_w(j, j % NBUF)


NPHASE = 4
SPLIT_ROWS = E_PAD // NPHASE
CPW_S = CPW // NPHASE


def _sc_gather(h_pad, idx2d, phase):
    mesh = plsc.VectorSubcoreMesh(core_axis_name="c", subcore_axis_name="s")
    kern = pl.kernel(
        functools.partial(_sc_gather_body, phase * (SPLIT_ROWS // CHUNK),
                          CPW_S, 24),
        out_type=jax.ShapeDtypeStruct((SPLIT_ROWS, D), f32),
        mesh=mesh,
        scratch_types=[
            pltpu.VMEM((CPW_S, CHUNK), i32),
            pltpu.VMEM((NBUF, CHUNK, D), f32),
            pltpu.SemaphoreType.DMA((NBUF,)),
            pltpu.SemaphoreType.DMA((NBUF,)),
        ],
    )
    return kern(h_pad, idx2d)


def _prolog_body(x_ref, wf_ref, bf_ref, h_ref, h16_ref):
    h = jnp.dot(x_ref[...], wf_ref[...], preferred_element_type=f32)
    h = jax.nn.gelu(h + bf_ref[...])
    h_ref[:, 0:H] = h
    h_ref[:, H:D] = jnp.zeros((h.shape[0], D - H), f32)
    h16_ref[...] = h.astype(bf16)


def _prolog(x, W_feat, b_feat):
    blk = 2000
    return pl.pallas_call(
        _prolog_body,
        grid=(N // blk,),
        in_specs=[
            pl.BlockSpec((blk, D), lambda b: (b, 0)),
            pl.BlockSpec((D, H), lambda b: (0, 0)),
            pl.BlockSpec((1, H), lambda b: (0, 0)),
        ],
        out_specs=[
            pl.BlockSpec((blk, D), lambda b: (b, 0)),
            pl.BlockSpec((blk, H), lambda b: (b, 0)),
        ],
        out_shape=[
            jax.ShapeDtypeStruct((N, D), f32),
            jax.ShapeDtypeStruct((N, H), bf16),
        ],
    )(x, W_feat, b_feat.reshape(1, H))


def _embed_body(et_ref, rc0_ref, rc1_ref, ET_ref, Wrc_ref, brc_ref, e_ref):
    etr = et_ref[0]
    ids48 = lax.broadcasted_iota(i32, (NETP, T), 0)
    OH = (ids48 == etr).astype(bf16)
    rc0 = rc0_ref[0].astype(bf16)
    rc1 = rc1_ref[0].astype(bf16)
    RC = jnp.concatenate([rc0, rc1, jnp.zeros((6, T), bf16)], axis=0)
    cdn = (((0,), (0,)), ((), ()))
    e_f = lax.dot_general(OH, ET_ref[...], cdn, preferred_element_type=f32)
    e_f = e_f + lax.dot_general(RC, Wrc_ref[...], cdn, preferred_element_type=f32)
    e_ref[...] = (e_f + brc_ref[...]).astype(bf16)


def _embed(et3, rc03, rc13, ET48, Wrc8, brc_r):
    return pl.pallas_call(
        _embed_body,
        grid=(G,),
        in_specs=[
            pl.BlockSpec((1, 1, T), lambda g: (g, 0, 0)),
            pl.BlockSpec((1, 1, T), lambda g: (g, 0, 0)),
            pl.BlockSpec((1, 1, T), lambda g: (g, 0, 0)),
            pl.BlockSpec((NETP, H), lambda g: (0, 0)),
            pl.BlockSpec((8, H), lambda g: (0, 0)),
            pl.BlockSpec((1, H), lambda g: (0, 0)),
        ],
        out_specs=pl.BlockSpec((T, H), lambda g: (g, 0)),
        out_shape=jax.ShapeDtypeStruct((E, H), bf16),
    )(et3, rc03, rc13, ET48, Wrc8, brc_r)


def _edge_body(bases_ref, hs_ref, e_ref, dst_ref, h16_ref,
               A192_ref, bA_ref, attn_ref, ones_ref, exp8_ref,
               z_in_ref, den_in_ref, z_ref, den_ref):
    g = pl.program_id(0)

    @pl.when(g == 0)
    def _():
        z_ref[...] = z_in_ref[...]
        den_ref[...] = den_in_ref[...]

    base = pl.multiple_of(bases_ref[g], 16)
    dstr = dst_ref[0]
    hs16 = hs_ref[:, 0:H].astype(bf16)
    e16 = e_ref[...]

    idsW = lax.broadcasted_iota(i32, (W, T), 0) + base
    PT = (idsW == dstr).astype(bf16)

    cdn = (((0,), (0,)), ((), ()))
    h_win = h16_ref[pl.ds(base, W), :]
    hd16 = lax.dot_general(PT, h_win, cdn, preferred_element_type=f32).astype(bf16)

    cat = jnp.concatenate([hs16, e16, hd16], axis=1)
    f = jnp.dot(cat, A192_ref[...], preferred_element_type=f32)
    f = f + bA_ref[...]
    f = jnp.where(f >= 0, f, 0.2 * f)

    fa16 = (f * attn_ref[...]).astype(bf16)
    lg = jnp.dot(fa16, ones_ref[...], preferred_element_type=f32)
    ex = jnp.exp(lg)
    ex16 = ex.astype(bf16)

    den_ref[pl.ds(base, W), :] += jnp.dot(PT, ex16, preferred_element_type=f32)

    exb = jnp.dot(ex16, exp8_ref[...], preferred_element_type=f32).astype(bf16)
    hs4 = jnp.concatenate([hs16, hs16, hs16, hs16], axis=1)
    p_cat = hs4 * exb
    z_ref[pl.ds(base, W), :] += jnp.dot(PT, p_cat, preferred_element_type=f32)


def _edge(bases_ph, hs_ph, e16, dst3, h16, A192l, bAl, attnl, ones_blk, exp8,
          z_in, den_in, tile_off, ntiles):
    grid_spec = pltpu.PrefetchScalarGridSpec(
        num_scalar_prefetch=1,
        grid=(ntiles,),
        in_specs=[
            pl.BlockSpec((T, D), lambda g, b: (g, 0)),
            pl.BlockSpec((T, H), lambda g, b: (g + tile_off, 0)),
            pl.BlockSpec((1, 1, T), lambda g, b: (g + tile_off, 0, 0)),
            pl.BlockSpec((N, H), lambda g, b: (0, 0)),
            pl.BlockSpec((3 * H, 4 * H), lambda g, b: (0, 0)),
            pl.BlockSpec((1, 4 * H), lambda g, b: (0, 0)),
            pl.BlockSpec((1, 4 * H), lambda g, b: (0, 0)),
            pl.BlockSpec((4 * H, 8), lambda g, b: (0, 0)),
            pl.BlockSpec((8, 4 * H), lambda g, b: (0, 0)),
            pl.BlockSpec((N, 4 * H), lambda g, b: (0, 0)),
            pl.BlockSpec((N, 8), lambda g, b: (0, 0)),
        ],
        out_specs=[
            pl.BlockSpec((N, 4 * H), lambda g, b: (0, 0)),
            pl.BlockSpec((N, 8), lambda g, b: (0, 0)),
        ],
    )
    return pl.pallas_call(
        _edge_body,
        grid_spec=grid_spec,
        out_shape=[jax.ShapeDtypeStruct((N, 4 * H), f32),
                   jax.ShapeDtypeStruct((N, 8), f32)],
    )(bases_ph, hs_ph, e16, dst3, h16, A192l, bAl, attnl, ones_blk, exp8,
      z_in, den_in)


def _node_body(z_ref, den_ref, h_ref, WW_ref,
               bm_ref, h_out_ref, h16_out_ref):
    u = None
    for k in range(HEADS):
        r = 1.0 / (den_ref[:, k:k + 1] + 1e-9)
        q = z_ref[:, k * H:(k + 1) * H] * r
        t = jnp.dot(q, WW_ref[k], preferred_element_type=f32)
        u = t if u is None else u + t
    hn = jax.nn.gelu(u + bm_ref[...])
    h = hn + h_ref[:, 0:H]
    h_out_ref[:, 0:H] = h
    h_out_ref[:, H:D] = jnp.zeros((h.shape[0], D - H), f32)
    h16_out_ref[...] = h.astype(bf16)


def _node(z, den, h, WWl, bm_r):
    blk = 2000
    return pl.pallas_call(
        _node_body,
        grid=(N // blk,),
        in_specs=[
            pl.BlockSpec((blk, 4 * H), lambda b: (b, 0)),
            pl.BlockSpec((blk, 8), lambda b: (b, 0)),
            pl.BlockSpec((blk, D), lambda b: (b, 0)),
            pl.BlockSpec((HEADS, H, H), lambda b: (0, 0, 0)),
            pl.BlockSpec((1, H), lambda b: (0, 0)),
        ],
        out_specs=[
            pl.BlockSpec((blk, D), lambda b: (b, 0)),
            pl.BlockSpec((blk, H), lambda b: (b, 0)),
        ],
        out_shape=[
            jax.ShapeDtypeStruct((N, D), f32),
            jax.ShapeDtypeStruct((N, H), bf16),
        ],
    )(z, den, h, WWl, bm_r)


_NF = D + (LAYERS + 1) * H


def _final_body(x_ref, h0_ref, h1_ref, h2_ref, h3_ref, h4_ref, out_ref,
                gmax_ref):
    p = pl.program_id(0)
    b = pl.program_id(1)
    hs = (h0_ref, h1_ref, h2_ref, h3_ref, h4_ref)

    @pl.when((p == 0) & (b == 0))
    def _():
        gmax_ref[...] = jnp.full_like(gmax_ref, -jnp.inf)

    @pl.when(p == 0)
    def _():
        m = jnp.max(x_ref[...], axis=0, keepdims=True)
        gmax_ref[0:1, 0:D] = jnp.maximum(gmax_ref[0:1, 0:D], m)
        for i, hr in enumerate(hs):
            lo = D + i * H
            m = jnp.max(hr[:, 0:H], axis=0, keepdims=True)
            gmax_ref[0:1, lo:lo + H] = jnp.maximum(gmax_ref[0:1, lo:lo + H], m)

    @pl.when(p == 1)
    def _():
        out_ref[:, 0:D] = x_ref[...]
        for i, hr in enumerate(hs):
            lo = D + i * H
            out_ref[:, lo:lo + H] = hr[:, 0:H]
        g = gmax_ref[0:1, 0:_NF]
        out_ref[:, _NF:2 * _NF] = jnp.broadcast_to(g, (x_ref.shape[0], _NF))


def _final(x, h0, h1, h2, h3, h4):
    blk = 2000
    return pl.pallas_call(
        _final_body,
        grid=(2, N // blk),
        in_specs=[pl.BlockSpec((blk, D), lambda p, b: (b, 0))] * 6,
        out_specs=pl.BlockSpec((blk, 2 * _NF), lambda p, b: (b, 0)),
        out_shape=jax.ShapeDtypeStruct((N, 2 * _NF), f32),
        scratch_shapes=[pltpu.VMEM((8, _NF), f32)],
    )(x, h0, h1, h2, h3, h4)


def kernel(x, edge_index, edge_type, rc_att, W_feat, b_feat, ET, Wrc, brc, A,
           bA, attn, Wn, Wm, bm):
    src = edge_index[0]
    dst = edge_index[1]

    dst_s, perm = lax.sort_key_val(dst, jnp.arange(E, dtype=i32))
    src_s = jnp.take(src, perm)
    et_s = jnp.take(edge_type, perm)
    rc_s = jnp.take(rc_att, perm, axis=0)

    bases = jnp.minimum((dst_s[::T] // 16) * 16, N - W).astype(i32)

    et3 = et_s.reshape(G, 1, T)
    dst3 = dst_s.reshape(G, 1, T)
    rc03 = rc_s[:, 0].reshape(G, 1, T)
    rc13 = rc_s[:, 1].reshape(G, 1, T)
    src_pad = jnp.concatenate([src_s, jnp.zeros(E_PAD - E, i32)])
    idx2d = src_pad.reshape(NPHASE * NW, CPW_S, CHUNK)

    ET48 = jnp.concatenate([ET, jnp.zeros((NETP - NET, H), f32)]).astype(bf16)
    Wrc8 = jnp.concatenate([Wrc, jnp.zeros((6, H), f32)]).astype(bf16)
    brc_r = brc.reshape(1, H)
    A192 = A.astype(bf16)
    bA_r = bA.reshape(LAYERS, 1, HEADS * H)
    attn_r = attn.reshape(LAYERS, 1, HEADS * H)
    col = jnp.arange(HEADS * H) // H
    ones_blk = (col[:, None] == jnp.arange(8)[None, :]).astype(bf16)
    exp8 = (jnp.arange(8)[:, None] == col[None, :]).astype(bf16)
    Wn4 = Wn.reshape(LAYERS, H, HEADS, H).transpose(0, 2, 1, 3)
    Wm4 = Wm.reshape(LAYERS, HEADS, H, H)
    WW = jnp.einsum("lkab,lkbc->lkac", Wn4, Wm4)
    bm_r = bm.reshape(LAYERS, 1, H)

    h, h16 = _prolog(x, W_feat, b_feat)
    e16 = _embed(et3, rc03, rc13, ET48, Wrc8, brc_r)
    feats = [h]

    GTP = SPLIT_ROWS // T
    z0 = jnp.zeros((N, HEADS * H), f32)
    d0 = jnp.zeros((N, 8), f32)
    for l in range(LAYERS):
        hs_ph = [_sc_gather(h, idx2d, p) for p in range(NPHASE)]
        z, den = z0, d0
        for p in range(NPHASE):
            off = p * GTP
            nt = min(GTP, G - off)
            z, den = _edge(bases[off:off + nt], hs_ph[p], e16, dst3, h16,
                           A192[l], bA_r[l], attn_r[l], ones_blk, exp8,
                           z, den, off, nt)
        h, h16 = _node(z, den, h, WW[l], bm_r[l])
        feats.append(h)

    return _final(x, *feats)
